# baseline (speedup 1.0000x reference)
import functools

import jax
import jax.numpy as jnp
from jax import lax
from jax.experimental import pallas as pl
from jax.experimental.pallas import tpu as pltpu
from jax.experimental.pallas import tpu_sc as plsc

_D = 32
_B = 16384
_K = 10
_NC = 2
_NS = 16
_NW = _NC * _NS
_BPW = _B // _NW
_CHUNK = 512

_sc_mesh = plsc.VectorSubcoreMesh(core_axis_name="c", subcore_axis_name="s")


@functools.partial(
    pl.kernel,
    out_type=(
        jax.ShapeDtypeStruct((_B, _D), jnp.float32),
        jax.ShapeDtypeStruct((_B, _D), jnp.float32),
        jax.ShapeDtypeStruct((_B * _K, _D), jnp.float32),
    ),
    mesh=_sc_mesh,
    scratch_types=(
        pltpu.VMEM((_CHUNK,), jnp.int32),
        pltpu.VMEM((_CHUNK, _D), jnp.float32),
        pltpu.SemaphoreType.DMA,
    ),
    compiler_params=pltpu.CompilerParams(use_tc_tiling_on_sc=False),
)
def _sc_gather(tidx, cidx, nidx, ttab, ctab, t_out, c_out, n_out,
               idx_v, rows_v, sem):
    wid = lax.axis_index("s") * _NC + lax.axis_index("c")
    base = wid * _BPW
    pltpu.sync_copy(tidx.at[pl.ds(base, _CHUNK)], idx_v)
    pltpu.async_copy(ttab.at[idx_v], rows_v, sem).wait()
    pltpu.sync_copy(rows_v, t_out.at[pl.ds(base, _CHUNK)])
    pltpu.sync_copy(cidx.at[pl.ds(base, _CHUNK)], idx_v)
    pltpu.async_copy(ctab.at[idx_v], rows_v, sem).wait()
    pltpu.sync_copy(rows_v, c_out.at[pl.ds(base, _CHUNK)])
    nbase = base * _K
    for j in range(_BPW * _K // _CHUNK):
        off = nbase + j * _CHUNK
        pltpu.sync_copy(nidx.at[pl.ds(off, _CHUNK)], idx_v)
        pltpu.async_copy(ctab.at[idx_v], rows_v, sem).wait()
        pltpu.sync_copy(rows_v, n_out.at[pl.ds(off, _CHUNK)])


_R = 2048


def _tc_loss_body(t_ref, c_ref, n_ref, o_ref):
    t = t_ref[...]
    pos = jnp.sum(t * c_ref[...], axis=1)
    acc = jnp.sum(jax.nn.log_sigmoid(pos))
    n = n_ref[...]
    for k in range(_K):
        s = jnp.sum(t * n[:, k * _D:(k + 1) * _D], axis=1)
        acc = acc + jnp.sum(jax.nn.log_sigmoid(-s))

    @pl.when(pl.program_id(0) == 0)
    def _():
        o_ref[...] = jnp.zeros_like(o_ref)

    o_ref[...] = o_ref[...] - acc


_tc_loss = pl.pallas_call(
    _tc_loss_body,
    grid=(_B // _R,),
    in_specs=[
        pl.BlockSpec((_R, _D), lambda i: (i, 0)),
        pl.BlockSpec((_R, _D), lambda i: (i, 0)),
        pl.BlockSpec((_R, _K * _D), lambda i: (i, 0)),
    ],
    out_specs=pl.BlockSpec((1, 1), lambda i: (0, 0)),
    out_shape=jax.ShapeDtypeStruct((1, 1), jnp.float32),
)


def kernel(target_idx, context_idx, neg_idx, emb_target_table, emb_context_table):
    tix = target_idx.astype(jnp.int32)
    cix = context_idx.astype(jnp.int32)
    nix = neg_idx.astype(jnp.int32).reshape(-1)
    t_rows, c_rows, n_rows = _sc_gather(tix, cix, nix,
                                        emb_target_table, emb_context_table)
    out = _tc_loss(t_rows, c_rows, n_rows.reshape(_B, _K * _D))
    return out.reshape(())
